# baseline (speedup 1.0000x reference)
import jax
import jax.numpy as jnp
from jax import lax
from jax.experimental import pallas as pl
from jax.experimental.pallas import tpu as pltpu
from jax.experimental.pallas import tpu_sc as plsc

N = 10000
D = 128
NC = 2
NS = 16
NW = NC * NS
L = 16
NP = 10240
KB = 128
EPW = 10240
NB = EPW // KB
EP = NW * EPW
NBUF = 2
GA = 1
RPT = NP // NS
RB = 2000

IR = 4

B0 = 160
B1 = 0
NBT = NS * (B0 + B1)

_mesh = plsc.VectorSubcoreMesh(core_axis_name="c", subcore_axis_name="s")


def _deg_body(dst_hbm, deg_out, didx, ones_v, zbuf, deg_sh):
    c = lax.axis_index("c")
    s = lax.axis_index("s")
    wid = s * NC + c
    pltpu.sync_copy(dst_hbm.at[wid], didx)
    for k in range(KB // L):
        ones_v[pl.ds(k * L, L)] = jnp.ones((L,), jnp.float32)
    for k in range(RPT // L):
        zbuf[pl.ds(k * L, L)] = jnp.zeros((L,), jnp.float32)
    pltpu.sync_copy(zbuf, deg_sh.at[pl.ds(s * RPT, RPT)])
    plsc.subcore_barrier()

    def body(j, carry):
        pltpu.sync_copy(ones_v, deg_sh.at[didx.at[j]], add=True)
        return carry

    lax.fori_loop(0, NB, body, 0)
    plsc.subcore_barrier()
    pltpu.sync_copy(deg_sh.at[pl.ds(s * RPT, RPT)],
                    deg_out.at[c, pl.ds(s * RPT, RPT)])


def _msg_body(src_hbm, dst_hbm, y_hbm, acc_out,
              sring, dring, gbuf, acc_sh, isem, gsem, ssem):
    c = lax.axis_index("c")
    s = lax.axis_index("s")
    base = jnp.where(c == 0, s * B0, NS * B0 + s * B1)
    nb_mine = jnp.where(c == 0, B0, B1)

    def zrow(r, carry):
        for k in range(D // L):
            gbuf[0, r, pl.ds(k * L, L)] = jnp.zeros((L,), jnp.float32)
        return carry

    lax.fori_loop(0, KB, zrow, 0)
    for i in range(RPT // KB):
        pltpu.sync_copy(gbuf.at[0], acc_sh.at[pl.ds(s * RPT + i * KB, KB)])
    plsc.subcore_barrier()

    def iload_start(j, r):
        pltpu.async_copy(src_hbm.at[base + j], sring.at[r], isem.at[r])
        pltpu.async_copy(dst_hbm.at[base + j], dring.at[r], isem.at[r])

    def iload_wait(r):
        pltpu.make_async_copy(src_hbm.at[base], sring.at[r],
                              isem.at[r]).wait()
        pltpu.make_async_copy(dst_hbm.at[base], dring.at[r],
                              isem.at[r]).wait()

    def gather_start(r, bb):
        pltpu.async_copy(y_hbm.at[sring.at[r]], gbuf.at[bb], gsem.at[bb])

    def gather_wait(bb):
        pltpu.make_async_copy(y_hbm.at[sring.at[0]], gbuf.at[bb],
                              gsem.at[bb]).wait()

    def scat_start(r, bb):
        pltpu.async_copy(gbuf.at[bb], acc_sh.at[dring.at[r]], ssem.at[bb],
                         add=True)

    def scat_wait(bb):
        pltpu.make_async_copy(gbuf.at[bb], acc_sh.at[dring.at[0]],
                              ssem.at[bb]).wait()

    @pl.when(nb_mine > 0)
    def _():
        iload_start(0, 0)
        iload_start(1, 1)
        iload_wait(0)
        gather_start(0, 0)

    def group(g, carry):
        for u in range(IR):
            j = g * IR + u
            b = u % NBUF

            @pl.when(j >= 1)
            def _():
                scat_wait((u + 1) % NBUF)

            @pl.when(j + 2 < nb_mine)
            def _():
                iload_start(j + 2, (u + 2) % IR)

            @pl.when(j + 1 < nb_mine)
            def _():
                iload_wait((u + 1) % IR)
                gather_start((u + 1) % IR, (u + 1) % NBUF)

            gather_wait(b)
            scat_start(u, b)
        return carry

    lax.fori_loop(0, nb_mine // IR, group, 0)

    @pl.when(nb_mine > 0)
    def _():
        scat_wait(1)
    plsc.subcore_barrier()
    pltpu.sync_copy(acc_sh.at[pl.ds(s * RPT, RPT)],
                    acc_out.at[c, pl.ds(s * RPT, RPT)])


def _mm_body(x_ref, w_ref, dg_ref, y_ref):
    dg = dg_ref[...]
    dinv = lax.rsqrt(dg[:, 0:1] + dg[:, 1:2] + 1.0)
    y_ref[...] = jnp.dot(x_ref[...], w_ref[...],
                         preferred_element_type=jnp.float32) * dinv


def _fin_body(acc_ref, y_ref, dg_ref, b_ref, o_ref):
    dg = dg_ref[...]
    dinv = lax.rsqrt(dg[:, 0:1] + dg[:, 1:2] + 1.0)
    tot = acc_ref[0] + acc_ref[1] + y_ref[...]
    o_ref[...] = tot * dinv + b_ref[...]


def kernel(x, edge_index, W, b):
    pad = EP - edge_index.shape[1]
    src_p = jnp.concatenate(
        [edge_index[0], jnp.zeros((pad,), jnp.int32)]).reshape(NBT, KB)
    dst_p = jnp.concatenate(
        [edge_index[1], jnp.full((pad,), N, jnp.int32)]).reshape(NBT, KB)

    deg_fn = pl.kernel(
        _deg_body,
        out_type=jax.ShapeDtypeStruct((NC, NP), jnp.float32),
        mesh=_mesh,
        scratch_types=[
            pltpu.VMEM((NB, KB), jnp.int32),
            pltpu.VMEM((KB,), jnp.float32),
            pltpu.VMEM((RPT,), jnp.float32),
            pltpu.VMEM_SHARED((NP,), jnp.float32),
        ],
    )
    deg = deg_fn(dst_p.reshape(NW, NB, KB))
    deg_t = deg.T

    y = pl.pallas_call(
        _mm_body,
        grid=(N // RB,),
        in_specs=[
            pl.BlockSpec((RB, D), lambda i: (i, 0)),
            pl.BlockSpec((D, D), lambda i: (0, 0)),
            pl.BlockSpec((RB, NC), lambda i: (i, 0)),
        ],
        out_specs=pl.BlockSpec((RB, D), lambda i: (i, 0)),
        out_shape=jax.ShapeDtypeStruct((N, D), jnp.float32),
    )(x, W, deg_t)

    msg_fn = pl.kernel(
        _msg_body,
        out_type=jax.ShapeDtypeStruct((NC, NP, D), jnp.float32),
        mesh=_mesh,
        scratch_types=[
            pltpu.VMEM((IR, KB), jnp.int32),
            pltpu.VMEM((IR, KB), jnp.int32),
            pltpu.VMEM((NBUF, KB, D), jnp.float32),
            pltpu.VMEM_SHARED((NP, D), jnp.float32),
            pltpu.SemaphoreType.DMA((IR,)),
            pltpu.SemaphoreType.DMA((NBUF,)),
            pltpu.SemaphoreType.DMA((NBUF,)),
        ],
    )
    acc = msg_fn(src_p, dst_p, y)

    out = pl.pallas_call(
        _fin_body,
        grid=(N // RB,),
        in_specs=[
            pl.BlockSpec((NC, RB, D), lambda i: (0, i, 0)),
            pl.BlockSpec((RB, D), lambda i: (i, 0)),
            pl.BlockSpec((RB, NC), lambda i: (i, 0)),
            pl.BlockSpec((1, D), lambda i: (0, 0)),
        ],
        out_specs=pl.BlockSpec((RB, D), lambda i: (i, 0)),
        out_shape=jax.ShapeDtypeStruct((N, D), jnp.float32),
    )(acc, y, deg_t, b.reshape(1, D))
    return out


# speedup vs baseline: 3.9555x; 3.9555x over previous
import jax
import jax.numpy as jnp
from jax import lax
from jax.experimental import pallas as pl
from jax.experimental.pallas import tpu as pltpu
from jax.experimental.pallas import tpu_sc as plsc

N = 10000
D = 128
NC = 2
NS = 16
NW = NC * NS
L = 16
NP = 10240
KB = 128
EPW = 10240
NB = EPW // KB
EP = NW * EPW
NBUF = 2
GA = 1
RPT = NP // NS
RB = 2000

IR = 4

B0 = 80
B1 = 80
NBT = NS * (B0 + B1)

_mesh = plsc.VectorSubcoreMesh(core_axis_name="c", subcore_axis_name="s")


def _deg_body(dst_hbm, deg_out, didx, ones_v, zbuf, deg_sh):
    c = lax.axis_index("c")
    s = lax.axis_index("s")
    wid = s * NC + c
    pltpu.sync_copy(dst_hbm.at[wid], didx)
    for k in range(KB // L):
        ones_v[pl.ds(k * L, L)] = jnp.ones((L,), jnp.float32)
    for k in range(RPT // L):
        zbuf[pl.ds(k * L, L)] = jnp.zeros((L,), jnp.float32)
    pltpu.sync_copy(zbuf, deg_sh.at[pl.ds(s * RPT, RPT)])
    plsc.subcore_barrier()

    def body(j, carry):
        pltpu.sync_copy(ones_v, deg_sh.at[didx.at[j]], add=True)
        return carry

    lax.fori_loop(0, NB, body, 0)
    plsc.subcore_barrier()
    pltpu.sync_copy(deg_sh.at[pl.ds(s * RPT, RPT)],
                    deg_out.at[c, pl.ds(s * RPT, RPT)])


def _msg_body(src_hbm, dst_hbm, y_hbm, acc_out,
              sring, dring, gbuf, acc_sh, isem, gsem, ssem):
    c = lax.axis_index("c")
    s = lax.axis_index("s")
    base = jnp.where(c == 0, s * B0, NS * B0 + s * B1)
    nb_mine = jnp.where(c == 0, B0, B1)

    def zrow(r, carry):
        for k in range(D // L):
            gbuf[0, r, pl.ds(k * L, L)] = jnp.zeros((L,), jnp.float32)
        return carry

    lax.fori_loop(0, KB, zrow, 0)
    for i in range(RPT // KB):
        pltpu.sync_copy(gbuf.at[0], acc_sh.at[pl.ds(s * RPT + i * KB, KB)])
    plsc.subcore_barrier()

    def iload_start(j, r):
        pltpu.async_copy(src_hbm.at[base + j], sring.at[r], isem.at[r])
        pltpu.async_copy(dst_hbm.at[base + j], dring.at[r], isem.at[r])

    def iload_wait(r):
        pltpu.make_async_copy(src_hbm.at[base], sring.at[r],
                              isem.at[r]).wait()
        pltpu.make_async_copy(dst_hbm.at[base], dring.at[r],
                              isem.at[r]).wait()

    def gather_start(r, bb):
        pltpu.async_copy(y_hbm.at[sring.at[r]], gbuf.at[bb], gsem.at[bb])

    def gather_wait(bb):
        pltpu.make_async_copy(y_hbm.at[sring.at[0]], gbuf.at[bb],
                              gsem.at[bb]).wait()

    def scat_start(r, bb):
        pltpu.async_copy(gbuf.at[bb], acc_sh.at[dring.at[r]], ssem.at[bb],
                         add=True)

    def scat_wait(bb):
        pltpu.make_async_copy(gbuf.at[bb], acc_sh.at[dring.at[0]],
                              ssem.at[bb]).wait()

    @pl.when(nb_mine > 0)
    def _():
        iload_start(0, 0)
        iload_start(1, 1)
        iload_wait(0)
        gather_start(0, 0)

    def group(g, carry):
        for u in range(IR):
            j = g * IR + u
            b = u % NBUF

            @pl.when(j >= 1)
            def _():
                scat_wait((u + 1) % NBUF)

            @pl.when(j + 2 < nb_mine)
            def _():
                iload_start(j + 2, (u + 2) % IR)

            @pl.when(j + 1 < nb_mine)
            def _():
                iload_wait((u + 1) % IR)
                gather_start((u + 1) % IR, (u + 1) % NBUF)

            gather_wait(b)
            scat_start(u, b)
        return carry

    lax.fori_loop(0, nb_mine // IR, group, 0)

    @pl.when(nb_mine > 0)
    def _():
        scat_wait(1)
    plsc.subcore_barrier()
    pltpu.sync_copy(acc_sh.at[pl.ds(s * RPT, RPT)],
                    acc_out.at[c, pl.ds(s * RPT, RPT)])


def _mm_body(x_ref, w_ref, dg_ref, y_ref):
    dg = dg_ref[...]
    dinv = lax.rsqrt(dg[:, 0:1] + dg[:, 1:2] + 1.0)
    y_ref[...] = jnp.dot(x_ref[...], w_ref[...],
                         preferred_element_type=jnp.float32) * dinv


def _fin_body(acc_ref, y_ref, dg_ref, b_ref, o_ref):
    dg = dg_ref[...]
    dinv = lax.rsqrt(dg[:, 0:1] + dg[:, 1:2] + 1.0)
    tot = acc_ref[0] + acc_ref[1] + y_ref[...]
    o_ref[...] = tot * dinv + b_ref[...]


def kernel(x, edge_index, W, b):
    pad = EP - edge_index.shape[1]
    pad_i = jnp.arange(pad, dtype=jnp.int32)
    src_p = jnp.concatenate(
        [edge_index[0], pad_i % N]).reshape(NBT, KB)
    dst_p = jnp.concatenate(
        [edge_index[1], N + pad_i % (NP - N)]).reshape(NBT, KB)

    deg_fn = pl.kernel(
        _deg_body,
        out_type=jax.ShapeDtypeStruct((NC, NP), jnp.float32),
        mesh=_mesh,
        scratch_types=[
            pltpu.VMEM((NB, KB), jnp.int32),
            pltpu.VMEM((KB,), jnp.float32),
            pltpu.VMEM((RPT,), jnp.float32),
            pltpu.VMEM_SHARED((NP,), jnp.float32),
        ],
    )
    deg = deg_fn(dst_p.reshape(NW, NB, KB))
    deg_t = deg.T

    y = pl.pallas_call(
        _mm_body,
        grid=(N // RB,),
        in_specs=[
            pl.BlockSpec((RB, D), lambda i: (i, 0)),
            pl.BlockSpec((D, D), lambda i: (0, 0)),
            pl.BlockSpec((RB, NC), lambda i: (i, 0)),
        ],
        out_specs=pl.BlockSpec((RB, D), lambda i: (i, 0)),
        out_shape=jax.ShapeDtypeStruct((N, D), jnp.float32),
    )(x, W, deg_t)

    msg_fn = pl.kernel(
        _msg_body,
        out_type=jax.ShapeDtypeStruct((NC, NP, D), jnp.float32),
        mesh=_mesh,
        scratch_types=[
            pltpu.VMEM((IR, KB), jnp.int32),
            pltpu.VMEM((IR, KB), jnp.int32),
            pltpu.VMEM((NBUF, KB, D), jnp.float32),
            pltpu.VMEM_SHARED((NP, D), jnp.float32),
            pltpu.SemaphoreType.DMA((IR,)),
            pltpu.SemaphoreType.DMA((NBUF,)),
            pltpu.SemaphoreType.DMA((NBUF,)),
        ],
    )
    acc = msg_fn(src_p, dst_p, y)

    out = pl.pallas_call(
        _fin_body,
        grid=(N // RB,),
        in_specs=[
            pl.BlockSpec((NC, RB, D), lambda i: (0, i, 0)),
            pl.BlockSpec((RB, D), lambda i: (i, 0)),
            pl.BlockSpec((RB, NC), lambda i: (i, 0)),
            pl.BlockSpec((1, D), lambda i: (0, 0)),
        ],
        out_specs=pl.BlockSpec((RB, D), lambda i: (i, 0)),
        out_shape=jax.ShapeDtypeStruct((N, D), jnp.float32),
    )(acc, y, deg_t, b.reshape(1, D))
    return out
